# baseline (speedup 1.0000x reference)
import jax
import jax.numpy as jnp
from jax.experimental import pallas as pl
from jax.experimental.pallas import tpu as pltpu

_VMEM_LIMIT_BYTES = 48 * 1024 * 1024


def _make_kernel(C, Ho, Wo):
    N = Ho * Wo

    def body(x_ref, w_ref, b_ref, o_ref, patch_ref):
        for ky in range(3):
            for kx in range(3):
                tap = ky * 3 + kx
                rp, ady = (1, -1) if ky == 0 else (ky - 1, 0)
                wp, adx = (1, -1) if kx == 0 else (kx - 1, 0)
                zp = rp * 2 + wp
                s = Wo * ady + adx
                n_lo = max(Wo if ky == 0 else 0, -s if s < 0 else 0)
                v = x_ref[0, zp * C:(zp + 1) * C, n_lo + s:N + s]
                if kx == 0:
                    idx = jax.lax.broadcasted_iota(
                        jnp.int32, (C, N - n_lo), 1) + n_lo
                    v = jnp.where((idx & (Wo - 1)) == 0, jnp.bfloat16(0), v)
                patch_ref[tap * C:(tap + 1) * C, n_lo:N] = v
                if n_lo:
                    patch_ref[tap * C:(tap + 1) * C, 0:n_lo] = jnp.zeros(
                        (C, n_lo), jnp.bfloat16)

        acc = (jnp.dot(w_ref[...], patch_ref[...],
                       preferred_element_type=jnp.float32)
               + b_ref[...])
        for oy in range(Ho):
            o_ref[0, :, oy, :] = acc[:, oy * Wo:(oy + 1) * Wo]

    return body


def kernel(x, conv_w, conv_b):
    B, C, H, W = x.shape
    OC = conv_w.shape[0]
    Ho, Wo = H // 2, W // 2
    N = Ho * Wo

    xps = x.reshape(B, C, Ho, 2, Wo, 2).transpose(0, 3, 5, 1, 2, 4)
    xps = xps.astype(jnp.bfloat16).reshape(B, 4 * C, N)

    w2 = conv_w.transpose(0, 2, 3, 1).reshape(OC, 9 * C).astype(jnp.bfloat16)
    b2 = conv_b.reshape(OC, 1).astype(jnp.float32)

    out = pl.pallas_call(
        _make_kernel(C, Ho, Wo),
        out_shape=jax.ShapeDtypeStruct((B, OC, Ho, Wo), jnp.float32),
        grid=(B,),
        in_specs=[
            pl.BlockSpec((1, 4 * C, N), lambda i: (i, 0, 0)),
            pl.BlockSpec((OC, 9 * C), lambda i: (0, 0)),
            pl.BlockSpec((OC, 1), lambda i: (0, 0)),
        ],
        out_specs=pl.BlockSpec((1, OC, Ho, Wo), lambda i: (i, 0, 0, 0)),
        scratch_shapes=[pltpu.VMEM((9 * C, N), jnp.bfloat16)],
        compiler_params=pltpu.CompilerParams(
            dimension_semantics=("parallel",),
            vmem_limit_bytes=_VMEM_LIMIT_BYTES),
    )(xps, w2, b2)
    return out


# speedup vs baseline: 1.5749x; 1.5749x over previous
import jax
import jax.numpy as jnp
from jax.experimental import pallas as pl
from jax.experimental.pallas import tpu as pltpu

_VMEM_LIMIT_BYTES = 48 * 1024 * 1024


def _make_kernel(C, Ho, Wo):
    N = Ho * Wo

    def body(x_ref, w_ref, b_ref, o_ref, patch_ref):
        for ky in range(3):
            for kx in range(3):
                tap = ky * 3 + kx
                rp, ady = (1, -1) if ky == 0 else (ky - 1, 0)
                wp, adx = (1, -1) if kx == 0 else (kx - 1, 0)
                zp = rp * 2 + wp
                s = Wo * ady + adx
                n_lo = max(Wo if ky == 0 else 0, -s if s < 0 else 0)
                v = x_ref[0, zp * C:(zp + 1) * C, n_lo + s:N + s]
                if kx == 0:
                    idx = jax.lax.broadcasted_iota(
                        jnp.int32, (C, N - n_lo), 1) + n_lo
                    v = jnp.where((idx & (Wo - 1)) == 0, jnp.bfloat16(0), v)
                patch_ref[tap * C:(tap + 1) * C, n_lo:N] = v
                if n_lo:
                    patch_ref[tap * C:(tap + 1) * C, 0:n_lo] = jnp.zeros(
                        (C, n_lo), jnp.bfloat16)

        o_ref[0] = (jnp.dot(w_ref[...], patch_ref[...],
                            preferred_element_type=jnp.float32)
                    + b_ref[...])

    return body


def kernel(x, conv_w, conv_b):
    B, C, H, W = x.shape
    OC = conv_w.shape[0]
    Ho, Wo = H // 2, W // 2
    N = Ho * Wo

    xps = x.astype(jnp.bfloat16).reshape(B, C, Ho, 2, Wo, 2)
    xps = xps.transpose(0, 3, 5, 1, 2, 4).reshape(B, 4 * C, N)

    w2 = conv_w.transpose(0, 2, 3, 1).reshape(OC, 9 * C).astype(jnp.bfloat16)
    b2 = conv_b.reshape(OC, 1).astype(jnp.float32)

    out = pl.pallas_call(
        _make_kernel(C, Ho, Wo),
        out_shape=jax.ShapeDtypeStruct((B, OC, N), jnp.float32),
        grid=(B,),
        in_specs=[
            pl.BlockSpec((1, 4 * C, N), lambda i: (i, 0, 0)),
            pl.BlockSpec((OC, 9 * C), lambda i: (0, 0)),
            pl.BlockSpec((OC, 1), lambda i: (0, 0)),
        ],
        out_specs=pl.BlockSpec((1, OC, N), lambda i: (i, 0, 0)),
        scratch_shapes=[pltpu.VMEM((9 * C, N), jnp.bfloat16)],
        compiler_params=pltpu.CompilerParams(
            dimension_semantics=("parallel",),
            vmem_limit_bytes=_VMEM_LIMIT_BYTES),
    )(xps, w2, b2)
    return out.reshape(B, OC, Ho, Wo)


# speedup vs baseline: 1.6387x; 1.0405x over previous
import jax
import jax.numpy as jnp
from jax.experimental import pallas as pl
from jax.experimental.pallas import tpu as pltpu

_VMEM_LIMIT_BYTES = 48 * 1024 * 1024
_ELEMS = 2


def _make_kernel(C, Ho, Wo):
    N = Ho * Wo

    def body(x_ref, w_ref, b_ref, o_ref, patch_ref):
        for e in range(_ELEMS):
            for ky in range(3):
                for kx in range(3):
                    tap = ky * 3 + kx
                    rp, ady = (1, -1) if ky == 0 else (ky - 1, 0)
                    wp, adx = (1, -1) if kx == 0 else (kx - 1, 0)
                    zp = rp * 2 + wp
                    s = Wo * ady + adx
                    n_lo = max(Wo if ky == 0 else 0, -s if s < 0 else 0)
                    v = x_ref[e, zp * C:(zp + 1) * C, n_lo + s:N + s]
                    if kx == 0:
                        idx = jax.lax.broadcasted_iota(
                            jnp.int32, (C, N - n_lo), 1) + n_lo
                        v = jnp.where((idx & (Wo - 1)) == 0, jnp.bfloat16(0), v)
                    patch_ref[tap * C:(tap + 1) * C,
                              e * N + n_lo:e * N + N] = v
                    if n_lo:
                        patch_ref[tap * C:(tap + 1) * C,
                                  e * N:e * N + n_lo] = jnp.zeros(
                                      (C, n_lo), jnp.bfloat16)

        acc = (jnp.dot(w_ref[...], patch_ref[...],
                       preferred_element_type=jnp.float32) + b_ref[...])
        for e in range(_ELEMS):
            o_ref[e] = acc[:, e * N:(e + 1) * N]

    return body


def kernel(x, conv_w, conv_b):
    B, C, H, W = x.shape
    OC = conv_w.shape[0]
    Ho, Wo = H // 2, W // 2
    N = Ho * Wo

    xps = x.astype(jnp.bfloat16).reshape(B, C, Ho, 2, Wo, 2)
    xps = xps.transpose(0, 3, 5, 1, 2, 4).reshape(B, 4 * C, N)

    w2 = conv_w.transpose(0, 2, 3, 1).reshape(OC, 9 * C).astype(jnp.bfloat16)
    b2 = conv_b.reshape(OC, 1).astype(jnp.float32)

    out = pl.pallas_call(
        _make_kernel(C, Ho, Wo),
        out_shape=jax.ShapeDtypeStruct((B, OC, N), jnp.float32),
        grid=(B // _ELEMS,),
        in_specs=[
            pl.BlockSpec((_ELEMS, 4 * C, N), lambda i: (i, 0, 0)),
            pl.BlockSpec((OC, 9 * C), lambda i: (0, 0)),
            pl.BlockSpec((OC, 1), lambda i: (0, 0)),
        ],
        out_specs=pl.BlockSpec((_ELEMS, OC, N), lambda i: (i, 0, 0)),
        scratch_shapes=[pltpu.VMEM((9 * C, _ELEMS * N), jnp.bfloat16)],
        compiler_params=pltpu.CompilerParams(
            dimension_semantics=("parallel",),
            vmem_limit_bytes=_VMEM_LIMIT_BYTES),
    )(xps, w2, b2)
    return out.reshape(B, OC, Ho, Wo)


# speedup vs baseline: 1.6812x; 1.0259x over previous
import jax
import jax.numpy as jnp
from jax.experimental import pallas as pl
from jax.experimental.pallas import tpu as pltpu

_VMEM_LIMIT_BYTES = 48 * 1024 * 1024
_ELEMS = 4


def _make_kernel(C, Ho, Wo):
    N = Ho * Wo

    def body(x_ref, w_ref, b_ref, o_ref, patch_ref):
        for e in range(_ELEMS):
            for ky in range(3):
                for kx in range(3):
                    tap = ky * 3 + kx
                    rp, ady = (1, -1) if ky == 0 else (ky - 1, 0)
                    wp, adx = (1, -1) if kx == 0 else (kx - 1, 0)
                    zp = rp * 2 + wp
                    s = Wo * ady + adx
                    n_lo = max(Wo if ky == 0 else 0, -s if s < 0 else 0)
                    v = x_ref[e, zp * C:(zp + 1) * C, n_lo + s:N + s]
                    if kx == 0:
                        idx = jax.lax.broadcasted_iota(
                            jnp.int32, (C, N - n_lo), 1) + n_lo
                        v = jnp.where((idx & (Wo - 1)) == 0, jnp.bfloat16(0), v)
                    patch_ref[tap * C:(tap + 1) * C,
                              e * N + n_lo:e * N + N] = v
                    if n_lo:
                        patch_ref[tap * C:(tap + 1) * C,
                                  e * N:e * N + n_lo] = jnp.zeros(
                                      (C, n_lo), jnp.bfloat16)

        acc = (jnp.dot(w_ref[...], patch_ref[...],
                       preferred_element_type=jnp.float32) + b_ref[...])
        for e in range(_ELEMS):
            o_ref[e] = acc[:, e * N:(e + 1) * N]

    return body


def kernel(x, conv_w, conv_b):
    B, C, H, W = x.shape
    OC = conv_w.shape[0]
    Ho, Wo = H // 2, W // 2
    N = Ho * Wo

    xps = x.astype(jnp.bfloat16).reshape(B, C, Ho, 2, Wo, 2)
    xps = xps.transpose(0, 3, 5, 1, 2, 4).reshape(B, 4 * C, N)

    w2 = conv_w.transpose(0, 2, 3, 1).reshape(OC, 9 * C).astype(jnp.bfloat16)
    b2 = conv_b.reshape(OC, 1).astype(jnp.float32)

    out = pl.pallas_call(
        _make_kernel(C, Ho, Wo),
        out_shape=jax.ShapeDtypeStruct((B, OC, N), jnp.float32),
        grid=(B // _ELEMS,),
        in_specs=[
            pl.BlockSpec((_ELEMS, 4 * C, N), lambda i: (i, 0, 0)),
            pl.BlockSpec((OC, 9 * C), lambda i: (0, 0)),
            pl.BlockSpec((OC, 1), lambda i: (0, 0)),
        ],
        out_specs=pl.BlockSpec((_ELEMS, OC, N), lambda i: (i, 0, 0)),
        scratch_shapes=[pltpu.VMEM((9 * C, _ELEMS * N), jnp.bfloat16)],
        compiler_params=pltpu.CompilerParams(
            dimension_semantics=("parallel",),
            vmem_limit_bytes=_VMEM_LIMIT_BYTES),
    )(xps, w2, b2)
    return out.reshape(B, OC, Ho, Wo)
